# baseline (speedup 1.0000x reference)
import functools

import jax
import jax.numpy as jnp
from jax import lax
from jax.experimental import pallas as pl
from jax.experimental.pallas import tpu as pltpu
from jax.experimental.pallas import tpu_sc as plsc

N_NODES = 50000
LANES = 16
N_PAD = 50048
QUARTER = N_PAD // 4
SUBQ = QUARTER // 2
CHUNK = 3200
CBLK = CHUNK // 128
FUSE = 4


def _sc_body(E, EP, NCH,
             dst_hbm, lgT_hbm, scB_hbm,
             out_hbm, part_hbm, fin_hbm,
             table, table_b, dst_buf, lg_buf, sc_buf, out_buf, red_a, red_b):
    c = lax.axis_index("c")
    s = lax.axis_index("s")
    head_local = s // 4
    head = c * 4 + head_local
    part = s % 4
    w = c * 16 + s

    def init_body(i, _):
        ninf = jnp.full((LANES,), -jnp.inf, jnp.float32)
        table[pl.ds(i * LANES, LANES)] = ninf
        table_b[pl.ds(i * LANES, LANES)] = ninf
        return 0
    lax.fori_loop(0, N_PAD // LANES, init_body, 0)

    def chunk1(ci, _):
        base = pl.multiple_of(part * EP + ci * CHUNK, 128)
        pltpu.sync_copy(dst_hbm.at[pl.ds(base, CHUNK)], dst_buf)
        pltpu.sync_copy(lgT_hbm.at[pl.ds(head * E + base, CHUNK)], lg_buf)

        tabsel = [table, table_b, table, table_b]

        def vec(j4, _):
            j0 = j4 * FUSE
            ds_ = [dst_buf[pl.ds((j0 + k) * LANES, LANES)] for k in range(FUSE)]
            vs = [lg_buf[pl.ds((j0 + k) * LANES, LANES)] for k in range(FUSE)]
            gs = tuple(plsc.load_gather(tabsel[k], [ds_[k]]) for k in range(FUSE))

            def cond(gc):
                m = vs[0] > gc[0]
                for k in range(1, FUSE):
                    m = m | (vs[k] > gc[k])
                return jnp.any(m)

            def wbody(gc):
                for k in range(FUSE):
                    plsc.store_scatter(tabsel[k], [ds_[k]], vs[k],
                                       mask=vs[k] > gc[k])
                return tuple(plsc.load_gather(tabsel[k], [ds_[k]])
                             for k in range(FUSE))

            lax.while_loop(cond, wbody, gs)
            return 0
        lax.fori_loop(0, CHUNK // LANES // FUSE, vec, 0)
        return 0
    lax.fori_loop(0, NCH, chunk1, 0)

    pltpu.sync_copy(table, part_hbm.at[pl.ds(w * N_PAD, N_PAD)])
    pltpu.sync_copy(table_b, part_hbm.at[pl.ds((32 + w) * N_PAD, N_PAD)])
    plsc.subcore_barrier()

    team = c * 16 + head_local * 4
    rows = [team + j for j in range(4)] + [32 + team + j for j in range(4)]
    for q2 in range(2):
        qoff = part * QUARTER + q2 * SUBQ
        pltpu.sync_copy(part_hbm.at[pl.ds(rows[0] * N_PAD + qoff, SUBQ)], red_a)
        for r in rows[1:]:
            pltpu.sync_copy(part_hbm.at[pl.ds(r * N_PAD + qoff, SUBQ)],
                            red_b)

            def mx_body(i, _):
                sl = pl.ds(i * LANES, LANES)
                red_a[sl] = jnp.maximum(red_a[sl], red_b[sl])
                return 0
            lax.fori_loop(0, SUBQ // LANES, mx_body, 0)
        pltpu.sync_copy(red_a, fin_hbm.at[pl.ds(head * N_PAD + qoff, SUBQ)])
    plsc.subcore_barrier()
    pltpu.sync_copy(fin_hbm.at[pl.ds(head * N_PAD, N_PAD)], table)

    def chunk3(ci, _):
        base = pl.multiple_of(part * EP + ci * CHUNK, 128)
        bblk = part * (EP // 128) + ci * CBLK
        pltpu.sync_copy(dst_hbm.at[pl.ds(base, CHUNK)], dst_buf)
        pltpu.sync_copy(lgT_hbm.at[pl.ds(head * E + base, CHUNK)], lg_buf)
        pltpu.sync_copy(scB_hbm.at[pl.ds(bblk, CBLK), head, :], sc_buf)

        @plsc.parallel_loop(0, CHUNK // LANES, unroll=4)
        def vec(j):
            sl = pl.ds(j * LANES, LANES)
            d = dst_buf[sl]
            mx = plsc.load_gather(table, [d])
            sc = sc_buf[j // 8, pl.ds((j % 8) * LANES, LANES)]
            out_buf[sl] = sc * jnp.exp(lg_buf[sl] - mx)
        pltpu.sync_copy(out_buf, out_hbm.at[pl.ds(head * E + base, CHUNK)])
        return 0
    lax.fori_loop(0, NCH, chunk3, 0)


def kernel(edge_index, logits, scale):
    E, H = scale.shape
    assert H == 8 and E % (4 * CHUNK) == 0 and E % 128 == 0
    EP = E // 4
    NCH = EP // CHUNK

    dst = edge_index[1]
    lgT = logits.transpose(1, 0, 2).reshape(H * E)
    scB = scale.reshape(E // 128, 128, H).transpose(0, 2, 1)

    mesh = plsc.VectorSubcoreMesh(core_axis_name="c", subcore_axis_name="s")
    params = pltpu.CompilerParams(needs_layout_passes=False)

    eT, _parts, _fin = pl.kernel(
        functools.partial(_sc_body, E, EP, NCH),
        out_type=(
            jax.ShapeDtypeStruct((H * E,), jnp.float32),
            jax.ShapeDtypeStruct((64 * N_PAD,), jnp.float32),
            jax.ShapeDtypeStruct((8 * N_PAD,), jnp.float32),
        ),
        mesh=mesh,
        compiler_params=params,
        scratch_types=[
            pltpu.VMEM((N_PAD,), jnp.float32),
            pltpu.VMEM((N_PAD,), jnp.float32),
            pltpu.VMEM((CHUNK,), jnp.int32),
            pltpu.VMEM((CHUNK,), jnp.float32),
            pltpu.VMEM((CBLK, 128), jnp.float32),
            pltpu.VMEM((CHUNK,), jnp.float32),
            pltpu.VMEM((SUBQ,), jnp.float32),
            pltpu.VMEM((SUBQ,), jnp.float32),
        ],
    )(dst, lgT, scB)

    return eT.reshape(1, H, E).transpose(2, 1, 0)


# speedup vs baseline: 1.1450x; 1.1450x over previous
import functools

import jax
import jax.numpy as jnp
from jax import lax
from jax.experimental import pallas as pl
from jax.experimental.pallas import tpu as pltpu
from jax.experimental.pallas import tpu_sc as plsc

N_NODES = 50000
LANES = 16
N_PAD = 50048
QUARTER = N_PAD // 4
SUBQ = QUARTER // 2
CHUNK = 3200
CBLK = CHUNK // 128
FUSE = 8


def _sc_body(E, EP, NCH,
             dst_hbm, lgT_hbm, scB_hbm,
             out_hbm, part_hbm, fin_hbm,
             table, table_b, dst_buf, lg_buf, sc_buf, out_buf, red_a, red_b):
    c = lax.axis_index("c")
    s = lax.axis_index("s")
    head_local = s // 4
    head = c * 4 + head_local
    part = s % 4
    w = c * 16 + s

    def init_body(i, _):
        ninf = jnp.full((LANES,), -jnp.inf, jnp.float32)
        table[pl.ds(i * LANES, LANES)] = ninf
        table_b[pl.ds(i * LANES, LANES)] = ninf
        return 0
    lax.fori_loop(0, N_PAD // LANES, init_body, 0)

    def chunk1(ci, _):
        base = pl.multiple_of(part * EP + ci * CHUNK, 128)
        pltpu.sync_copy(dst_hbm.at[pl.ds(base, CHUNK)], dst_buf)
        pltpu.sync_copy(lgT_hbm.at[pl.ds(head * E + base, CHUNK)], lg_buf)

        tabsel = [table, table_b] * (FUSE // 2)

        def vec(j4, _):
            j0 = j4 * FUSE
            ds_ = [dst_buf[pl.ds((j0 + k) * LANES, LANES)] for k in range(FUSE)]
            vs = [lg_buf[pl.ds((j0 + k) * LANES, LANES)] for k in range(FUSE)]
            gs = tuple(plsc.load_gather(tabsel[k], [ds_[k]]) for k in range(FUSE))

            def cond(gc):
                m = vs[0] > gc[0]
                for k in range(1, FUSE):
                    m = m | (vs[k] > gc[k])
                return jnp.any(m)

            def wbody(gc):
                for k in range(FUSE):
                    plsc.store_scatter(tabsel[k], [ds_[k]], vs[k],
                                       mask=vs[k] > gc[k])
                return tuple(plsc.load_gather(tabsel[k], [ds_[k]])
                             for k in range(FUSE))

            lax.while_loop(cond, wbody, gs)
            return 0
        lax.fori_loop(0, CHUNK // LANES // FUSE, vec, 0)
        return 0
    lax.fori_loop(0, NCH, chunk1, 0)

    pltpu.sync_copy(table, part_hbm.at[pl.ds(w * N_PAD, N_PAD)])
    pltpu.sync_copy(table_b, part_hbm.at[pl.ds((32 + w) * N_PAD, N_PAD)])
    plsc.subcore_barrier()

    team = c * 16 + head_local * 4
    rows = [team + j for j in range(4)] + [32 + team + j for j in range(4)]
    for q2 in range(2):
        qoff = part * QUARTER + q2 * SUBQ
        pltpu.sync_copy(part_hbm.at[pl.ds(rows[0] * N_PAD + qoff, SUBQ)], red_a)
        for r in rows[1:]:
            pltpu.sync_copy(part_hbm.at[pl.ds(r * N_PAD + qoff, SUBQ)],
                            red_b)

            def mx_body(i, _):
                sl = pl.ds(i * LANES, LANES)
                red_a[sl] = jnp.maximum(red_a[sl], red_b[sl])
                return 0
            lax.fori_loop(0, SUBQ // LANES, mx_body, 0)
        pltpu.sync_copy(red_a, fin_hbm.at[pl.ds(head * N_PAD + qoff, SUBQ)])
    plsc.subcore_barrier()
    pltpu.sync_copy(fin_hbm.at[pl.ds(head * N_PAD, N_PAD)], table)

    def chunk3(ci, _):
        base = pl.multiple_of(part * EP + ci * CHUNK, 128)
        bblk = part * (EP // 128) + ci * CBLK
        pltpu.sync_copy(dst_hbm.at[pl.ds(base, CHUNK)], dst_buf)
        pltpu.sync_copy(lgT_hbm.at[pl.ds(head * E + base, CHUNK)], lg_buf)
        pltpu.sync_copy(scB_hbm.at[pl.ds(bblk, CBLK), head, :], sc_buf)

        @plsc.parallel_loop(0, CHUNK // LANES, unroll=4)
        def vec(j):
            sl = pl.ds(j * LANES, LANES)
            d = dst_buf[sl]
            mx = plsc.load_gather(table, [d])
            sc = sc_buf[j // 8, pl.ds((j % 8) * LANES, LANES)]
            out_buf[sl] = sc * jnp.exp(lg_buf[sl] - mx)
        pltpu.sync_copy(out_buf, out_hbm.at[pl.ds(head * E + base, CHUNK)])
        return 0
    lax.fori_loop(0, NCH, chunk3, 0)


def kernel(edge_index, logits, scale):
    E, H = scale.shape
    assert H == 8 and E % (4 * CHUNK) == 0 and E % 128 == 0
    EP = E // 4
    NCH = EP // CHUNK

    dst = edge_index[1]
    lgT = logits.transpose(1, 0, 2).reshape(H * E)
    scB = scale.reshape(E // 128, 128, H).transpose(0, 2, 1)

    mesh = plsc.VectorSubcoreMesh(core_axis_name="c", subcore_axis_name="s")
    params = pltpu.CompilerParams(needs_layout_passes=False)

    eT, _parts, _fin = pl.kernel(
        functools.partial(_sc_body, E, EP, NCH),
        out_type=(
            jax.ShapeDtypeStruct((H * E,), jnp.float32),
            jax.ShapeDtypeStruct((64 * N_PAD,), jnp.float32),
            jax.ShapeDtypeStruct((8 * N_PAD,), jnp.float32),
        ),
        mesh=mesh,
        compiler_params=params,
        scratch_types=[
            pltpu.VMEM((N_PAD,), jnp.float32),
            pltpu.VMEM((N_PAD,), jnp.float32),
            pltpu.VMEM((CHUNK,), jnp.int32),
            pltpu.VMEM((CHUNK,), jnp.float32),
            pltpu.VMEM((CBLK, 128), jnp.float32),
            pltpu.VMEM((CHUNK,), jnp.float32),
            pltpu.VMEM((SUBQ,), jnp.float32),
            pltpu.VMEM((SUBQ,), jnp.float32),
        ],
    )(dst, lgT, scB)

    return eT.reshape(1, H, E).transpose(2, 1, 0)


# speedup vs baseline: 1.1840x; 1.0341x over previous
import functools

import jax
import jax.numpy as jnp
from jax import lax
from jax.experimental import pallas as pl
from jax.experimental.pallas import tpu as pltpu
from jax.experimental.pallas import tpu_sc as plsc

N_NODES = 50000
LANES = 16
N_PAD = 50048
QUARTER = N_PAD // 4
SUBQ = QUARTER // 2
CHUNK = 3200
CBLK = CHUNK // 128
FUSE = 10


def _sc_body(E, EP, NCH,
             dst_hbm, lgT_hbm, scB_hbm,
             out_hbm, part_hbm, fin_hbm,
             table, table_b, dst_buf, lg_buf, sc_buf, out_buf, red_a, red_b):
    c = lax.axis_index("c")
    s = lax.axis_index("s")
    head_local = s // 4
    head = c * 4 + head_local
    part = s % 4
    w = c * 16 + s

    def init_body(i, _):
        ninf = jnp.full((LANES,), -jnp.inf, jnp.float32)
        table[pl.ds(i * LANES, LANES)] = ninf
        table_b[pl.ds(i * LANES, LANES)] = ninf
        return 0
    lax.fori_loop(0, N_PAD // LANES, init_body, 0)

    def chunk1(ci, _):
        base = pl.multiple_of(part * EP + ci * CHUNK, 128)
        pltpu.sync_copy(dst_hbm.at[pl.ds(base, CHUNK)], dst_buf)
        pltpu.sync_copy(lgT_hbm.at[pl.ds(head * E + base, CHUNK)], lg_buf)

        tabsel = [table, table_b] * (FUSE // 2)

        def vec(j4, _):
            j0 = j4 * FUSE
            ds_ = [dst_buf[pl.ds((j0 + k) * LANES, LANES)] for k in range(FUSE)]
            vs = [lg_buf[pl.ds((j0 + k) * LANES, LANES)] for k in range(FUSE)]
            gs = tuple(plsc.load_gather(tabsel[k], [ds_[k]]) for k in range(FUSE))

            def cond(gc):
                m = vs[0] > gc[0]
                for k in range(1, FUSE):
                    m = m | (vs[k] > gc[k])
                return jnp.any(m)

            def wbody(gc):
                for k in range(FUSE):
                    plsc.store_scatter(tabsel[k], [ds_[k]], vs[k],
                                       mask=vs[k] > gc[k])
                return tuple(plsc.load_gather(tabsel[k], [ds_[k]])
                             for k in range(FUSE))

            lax.while_loop(cond, wbody, gs)
            return 0
        lax.fori_loop(0, CHUNK // LANES // FUSE, vec, 0)
        return 0
    lax.fori_loop(0, NCH, chunk1, 0)

    pltpu.sync_copy(table, part_hbm.at[pl.ds(w * N_PAD, N_PAD)])
    pltpu.sync_copy(table_b, part_hbm.at[pl.ds((32 + w) * N_PAD, N_PAD)])
    plsc.subcore_barrier()

    team = c * 16 + head_local * 4
    rows = [team + j for j in range(4)] + [32 + team + j for j in range(4)]
    for q2 in range(2):
        qoff = part * QUARTER + q2 * SUBQ
        pltpu.sync_copy(part_hbm.at[pl.ds(rows[0] * N_PAD + qoff, SUBQ)], red_a)
        for r in rows[1:]:
            pltpu.sync_copy(part_hbm.at[pl.ds(r * N_PAD + qoff, SUBQ)],
                            red_b)

            def mx_body(i, _):
                sl = pl.ds(i * LANES, LANES)
                red_a[sl] = jnp.maximum(red_a[sl], red_b[sl])
                return 0
            lax.fori_loop(0, SUBQ // LANES, mx_body, 0)
        pltpu.sync_copy(red_a, fin_hbm.at[pl.ds(head * N_PAD + qoff, SUBQ)])
    plsc.subcore_barrier()
    pltpu.sync_copy(fin_hbm.at[pl.ds(head * N_PAD, N_PAD)], table)

    def chunk3(ci, _):
        base = pl.multiple_of(part * EP + ci * CHUNK, 128)
        bblk = part * (EP // 128) + ci * CBLK
        pltpu.sync_copy(dst_hbm.at[pl.ds(base, CHUNK)], dst_buf)
        pltpu.sync_copy(lgT_hbm.at[pl.ds(head * E + base, CHUNK)], lg_buf)
        pltpu.sync_copy(scB_hbm.at[pl.ds(bblk, CBLK), head, :], sc_buf)

        @plsc.parallel_loop(0, CHUNK // LANES, unroll=4)
        def vec(j):
            sl = pl.ds(j * LANES, LANES)
            d = dst_buf[sl]
            mx = plsc.load_gather(table, [d])
            sc = sc_buf[j // 8, pl.ds((j % 8) * LANES, LANES)]
            out_buf[sl] = sc * jnp.exp(lg_buf[sl] - mx)
        pltpu.sync_copy(out_buf, out_hbm.at[pl.ds(head * E + base, CHUNK)])
        return 0
    lax.fori_loop(0, NCH, chunk3, 0)


def kernel(edge_index, logits, scale):
    E, H = scale.shape
    assert H == 8 and E % (4 * CHUNK) == 0 and E % 128 == 0
    EP = E // 4
    NCH = EP // CHUNK

    dst = edge_index[1]
    lgT = logits.transpose(1, 0, 2).reshape(H * E)
    scB = scale.reshape(E // 128, 128, H).transpose(0, 2, 1)

    mesh = plsc.VectorSubcoreMesh(core_axis_name="c", subcore_axis_name="s")
    params = pltpu.CompilerParams(needs_layout_passes=False)

    eT, _parts, _fin = pl.kernel(
        functools.partial(_sc_body, E, EP, NCH),
        out_type=(
            jax.ShapeDtypeStruct((H * E,), jnp.float32),
            jax.ShapeDtypeStruct((64 * N_PAD,), jnp.float32),
            jax.ShapeDtypeStruct((8 * N_PAD,), jnp.float32),
        ),
        mesh=mesh,
        compiler_params=params,
        scratch_types=[
            pltpu.VMEM((N_PAD,), jnp.float32),
            pltpu.VMEM((N_PAD,), jnp.float32),
            pltpu.VMEM((CHUNK,), jnp.int32),
            pltpu.VMEM((CHUNK,), jnp.float32),
            pltpu.VMEM((CBLK, 128), jnp.float32),
            pltpu.VMEM((CHUNK,), jnp.float32),
            pltpu.VMEM((SUBQ,), jnp.float32),
            pltpu.VMEM((SUBQ,), jnp.float32),
        ],
    )(dst, lgT, scB)

    return eT.reshape(1, H, E).transpose(2, 1, 0)
